# baseline (speedup 1.0000x reference)
import functools

import jax
import jax.numpy as jnp
from jax.experimental import pallas as pl
from jax.experimental.pallas import tpu as pltpu

_NUM_EMBEDDINGS = 512
_DIM = 32
_COMMITMENT_COST = 0.25
_BLOCK = 4000


def _vq_block_kernel(h_ref, cbneg2_ref, cb_ref, cc_ref, hh_ref,
                     q_ref, idx_ref, loss_ref):
    h = h_ref[...]
    t2 = jax.lax.dot_general(
        cbneg2_ref[...], h, (((1,), (1,)), ((), ())),
        preferred_element_type=jnp.float32)
    d = (hh_ref[0] + cc_ref[...]) + t2
    dmin = jnp.min(d, axis=0, keepdims=True)
    iota_s = jax.lax.broadcasted_iota(jnp.int32, d.shape, 0).astype(jnp.float32)
    idx_f = jnp.min(jnp.where(d <= dmin, iota_s, float(_NUM_EMBEDDINGS)),
                    axis=0, keepdims=True)
    onehot = (iota_s == idx_f).astype(jnp.float32)
    q = jax.lax.dot_general(
        onehot, cb_ref[...], (((0,), (0,)), ((), ())),
        preferred_element_type=jnp.float32)
    q_ref[...] = q
    idx_ref[...] = idx_f.astype(jnp.int32)[None]
    loss_ref[...] = jnp.sum(dmin, axis=1, keepdims=True)[None]


@functools.partial(jax.jit, static_argnames=())
def kernel(h_v_k, codebook):
    n, d = h_v_k.shape
    e = codebook.shape[0]
    cc = jnp.sum(codebook * codebook, axis=1)[:, None]
    grid = n // _BLOCK
    hh = jnp.sum(h_v_k * h_v_k, axis=1).reshape(grid, 1, _BLOCK)
    q, idx, loss = pl.pallas_call(
        _vq_block_kernel,
        grid=(grid,),
        in_specs=[
            pl.BlockSpec((_BLOCK, d), lambda i: (i, 0)),
            pl.BlockSpec((e, d), lambda i: (0, 0)),
            pl.BlockSpec((e, d), lambda i: (0, 0)),
            pl.BlockSpec((e, 1), lambda i: (0, 0)),
            pl.BlockSpec((1, 1, _BLOCK), lambda i: (i, 0, 0)),
        ],
        out_specs=[
            pl.BlockSpec((_BLOCK, d), lambda i: (i, 0)),
            pl.BlockSpec((1, 1, _BLOCK), lambda i: (i, 0, 0)),
            pl.BlockSpec((1, 1, 1), lambda i: (i, 0, 0)),
        ],
        out_shape=[
            jax.ShapeDtypeStruct((n, d), jnp.float32),
            jax.ShapeDtypeStruct((grid, 1, _BLOCK), jnp.int32),
            jax.ShapeDtypeStruct((grid, 1, 1), jnp.float32),
        ],
        compiler_params=pltpu.CompilerParams(
            dimension_semantics=("parallel",)),
    )(h_v_k, codebook * (-2.0), codebook, cc, hh)
    total_loss = jnp.sum(loss) * ((1.0 + _COMMITMENT_COST) / (n * d))
    return (q, idx.reshape(n), total_loss)
